# Initial kernel scaffold; baseline (speedup 1.0000x reference)
import jax
import jax.numpy as jnp
from jax.experimental import pallas as pl


def kernel(logits, k):
    raise NotImplementedError("write your pallas kernel here")


# speedup vs baseline: 2.3526x; 2.3526x over previous
import functools

import jax
import jax.numpy as jnp
from jax import lax
from jax.experimental import pallas as pl
from jax.experimental.pallas import tpu as pltpu
from jax.experimental.pallas import tpu_sc as plsc

B = 64
V = 1000000
K = 50
L = 16
NC, NS = 2, 16
ROWS_PER_W = B // (NC * NS)
W = 10000
NWIN = V // W
VREGS_PER_WIN = W // L
SLOTS = 256
NACC = SLOTS // L
CAP = 512
IDX_FILL = 2 ** 30

NEGINF = float("-inf")


def _neg16():
    return jnp.full((L,), NEGINF, jnp.float32)


def _accumulate_window(buf, acc):
    def body(i, acc):
        base = i * (NACC * L)
        new = []
        for j in range(NACC):
            v = buf[pl.ds(base + j * L, L)]
            new.append(jnp.maximum(acc[j], v))
        return tuple(new)
    acc = lax.fori_loop(0, VREGS_PER_WIN // NACC, body, acc)
    v = buf[pl.ds((VREGS_PER_WIN - 1) * L, L)]
    return (jnp.maximum(acc[0], v),) + acc[1:]


def _threshold(acc):
    def body(_, carry):
        vs, _thr = carry[:-1], carry[-1]
        m = vs[0]
        for j in range(1, NACC):
            m = jnp.maximum(m, vs[j])
        s = lax.sort(m)[L - 1]
        ms = jnp.full((L,), s, jnp.float32)
        vs2 = tuple(jnp.where(v == ms, _neg16(), v) for v in vs)
        return vs2 + (s,)
    out = lax.fori_loop(0, K, body, acc + (jnp.float32(0.0),))
    return out[-1]


def _filter_window(buf, woff, thr_s, cnt, cv, ci):
    thr = jnp.full((L,), thr_s, jnp.float32)
    iota = lax.iota(jnp.int32, L)

    def append_one(vj, base_idx, c):
        mask = vj >= thr
        idx = iota + base_idx
        cb = jnp.minimum(c, CAP - L)
        plsc.store_compressed(cv.at[pl.ds(cb, L)], vj, mask=mask)
        plsc.store_compressed(ci.at[pl.ds(cb, L)], idx, mask=mask)
        n = plsc.all_reduce_population_count(mask)
        return c + n[0]

    GRP = 8

    def grp(gi, c):
        base = gi * (GRP * L)
        vs = [buf[pl.ds(base + j * L, L)] for j in range(GRP)]
        gm = vs[0]
        for j in range(1, GRP):
            gm = jnp.maximum(gm, vs[j])
        hit = jnp.any(gm >= thr)

        def do(c2):
            for j in range(GRP):
                c2 = append_one(vs[j], woff + base + j * L, c2)
            return c2
        return lax.cond(hit, do, lambda c2: c2, c)

    cnt = lax.fori_loop(0, VREGS_PER_WIN // GRP, grp, cnt)
    tb = (VREGS_PER_WIN - 1) * L
    v = buf[pl.ds(tb, L)]
    hit = jnp.any(v >= thr)
    cnt = lax.cond(hit, lambda c: append_one(v, woff + tb, c), lambda c: c, cnt)
    return cnt


def _sc_body(logits_hbm, candv_hbm, candi_hbm, buf0, cv, ci):
    wid = lax.axis_index("s") * NC + lax.axis_index("c")
    for r in range(ROWS_PER_W):
        row = wid * ROWS_PER_W + r
        rbase = row * V

        def wbody(w, acc):
            pltpu.sync_copy(logits_hbm.at[pl.ds(rbase + w * W, W)], buf0)
            return _accumulate_window(buf0, acc)
        acc = lax.fori_loop(0, NWIN, wbody, (_neg16(),) * NACC)
        thr = _threshold(acc)

        for j in range(CAP // L):
            cv[pl.ds(j * L, L)] = _neg16()
            ci[pl.ds(j * L, L)] = jnp.full((L,), IDX_FILL, jnp.int32)

        def wbody2(w, cnt):
            pltpu.sync_copy(logits_hbm.at[pl.ds(rbase + w * W, W)], buf0)
            return _filter_window(buf0, w * W, thr, cnt, cv, ci)
        lax.fori_loop(0, NWIN, wbody2, jnp.int32(0))

        pltpu.sync_copy(cv, candv_hbm.at[pl.ds(row * CAP, CAP)])
        pltpu.sync_copy(ci, candi_hbm.at[pl.ds(row * CAP, CAP)])


@jax.jit
def _sc_topk_candidates(logits):
    mesh = plsc.VectorSubcoreMesh(core_axis_name="c", subcore_axis_name="s",
                                  num_cores=NC, num_subcores=NS)
    f = pl.kernel(
        _sc_body,
        out_type=(jax.ShapeDtypeStruct((B * CAP,), jnp.float32),
                  jax.ShapeDtypeStruct((B * CAP,), jnp.int32)),
        mesh=mesh,
        scratch_types=(
            pltpu.VMEM((W,), jnp.float32),
            pltpu.VMEM((CAP,), jnp.float32),
            pltpu.VMEM((CAP,), jnp.int32),
        ),
        compiler_params=pltpu.CompilerParams(needs_layout_passes=False),
    )
    candv, candi = f(logits.reshape(B * V))
    return candv.reshape(B, CAP), candi.reshape(B, CAP)


def _tc_body(candv_ref, candi_ref, g_ref, probs_ref, samples_ref):
    cv = candv_ref[...]
    ci = candi_ref[...]
    colk = lax.broadcasted_iota(jnp.int32, (B, K), 1)

    def body(t, carry):
        cv, tv, ti = carry
        m = jnp.max(cv, axis=1, keepdims=True)
        eq = cv == m
        isel = jnp.min(jnp.where(eq, ci, IDX_FILL), axis=1, keepdims=True)
        cv = jnp.where(eq & (ci == isel), NEGINF, cv)
        tv = jnp.where(colk == t, m, tv)
        ti = jnp.where(colk == t, isel, ti)
        return cv, tv, ti

    _, tv, ti = lax.fori_loop(
        0, K, body,
        (cv, jnp.zeros((B, K), jnp.float32), jnp.zeros((B, K), jnp.int32)))

    mx = jnp.max(tv, axis=1, keepdims=True)
    e = jnp.exp(tv - mx)
    probs = e / jnp.sum(e, axis=1, keepdims=True)
    probs_ref[...] = probs

    score = jnp.log(probs + 1e-20) + g_ref[...]
    smax = jnp.max(score, axis=1, keepdims=True)
    sel = jnp.min(jnp.where(score == smax, colk, IDX_FILL),
                  axis=1, keepdims=True)
    samples_ref[...] = jnp.sum(jnp.where(colk == sel, ti, 0),
                               axis=1, keepdims=True)


def kernel(logits, k):
    candv, candi = _sc_topk_candidates(logits)
    skey = jax.random.fold_in(jax.random.key(0), 1)
    u = jax.random.uniform(skey, (B, K), dtype=jnp.float32)
    g = -jnp.log(-jnp.log(u + 1e-20) + 1e-20)
    probs, samples = pl.pallas_call(
        _tc_body,
        out_shape=(jax.ShapeDtypeStruct((B, K), jnp.float32),
                   jax.ShapeDtypeStruct((B, 1), jnp.int32)),
    )(candv, candi, g)
    return probs, samples[:, 0]


# speedup vs baseline: 21.9906x; 9.3474x over previous
import jax
import jax.numpy as jnp
from jax import lax
from jax.experimental import pallas as pl
from jax.experimental.pallas import tpu as pltpu
from jax.experimental.pallas import tpu_sc as plsc

B = 64
V = 1000000
K = 50
L = 16
NC, NS = 2, 16
QS = 4
WC = 4096
NW_Q = 61
SHARD = NW_Q * WC
TAIL = V - QS * SHARD
TAILV = TAIL // L
SLOTS = 128
NACC = SLOTS // L
CAPL = 256
CAPT = QS * CAPL
IDX_FILL = 2 ** 30

NEGINF = float("-inf")


def _neg16():
    return jnp.full((L,), NEGINF, jnp.float32)


def _sc_body(logits_hbm, candv_hbm, candi_hbm,
             buf0, buf1, slotacc, cv, ci, thr_smem, cnt_smem, sem0, sem1):
    wid = lax.axis_index("s") * NC + lax.axis_index("c")
    blk = wid // QS
    q = wid % QS
    blk8 = pl.multiple_of(blk * 8, 8)
    qb = q * SHARD

    def src(w):
        off = pl.multiple_of(qb + w * WC, 128)
        return logits_hbm.at[pl.ds(blk8, 8), pl.ds(off, WC)]

    def run_windows(proc):
        pltpu.async_copy(src(0), buf0, sem0)

        def pair(p, _):
            w0 = 2 * p
            pltpu.async_copy(src(w0 + 1), buf1, sem1)
            pltpu.make_async_copy(src(w0), buf0, sem0).wait()
            proc(buf0, w0)
            pltpu.async_copy(src(w0 + 2), buf0, sem0)
            pltpu.make_async_copy(src(w0 + 1), buf1, sem1).wait()
            proc(buf1, w0 + 1)
            return 0

        lax.fori_loop(0, (NW_Q - 1) // 2, pair, 0)
        pltpu.make_async_copy(src(NW_Q - 1), buf0, sem0).wait()
        proc(buf0, NW_Q - 1)

    for r8 in range(8):
        for j in range(NACC):
            slotacc[r8, pl.ds(j * L, L)] = _neg16()

    def procA(buf, w):
        for r8 in range(8):
            acc = tuple(slotacc[r8, pl.ds(j * L, L)] for j in range(NACC))

            def body(ii, acc):
                base = ii * (NACC * L)
                return tuple(
                    jnp.maximum(acc[j], buf[r8, pl.ds(base + j * L, L)])
                    for j in range(NACC))

            acc = lax.fori_loop(0, WC // (NACC * L), body, acc)
            for j in range(NACC):
                slotacc[r8, pl.ds(j * L, L)] = acc[j]

    run_windows(procA)

    for r8 in range(8):
        acc = tuple(slotacc[r8, pl.ds(j * L, L)] for j in range(NACC))

        def tbody(_, carry):
            vs = carry[:-1]
            m = vs[0]
            for j in range(1, NACC):
                m = jnp.maximum(m, vs[j])
            s = lax.sort(m)[L - 1]
            ms = jnp.full((L,), s, jnp.float32)
            vs2 = tuple(jnp.where(v == ms, _neg16(), v) for v in vs)
            return vs2 + (s,)

        out = lax.fori_loop(0, K, tbody, acc + (jnp.float32(0.0),))
        thr_smem[r8] = out[-1]

    for r8 in range(8):
        cnt_smem[r8] = jnp.int32(0)
        for j in range(CAPL // L):
            cv[r8, pl.ds(j * L, L)] = _neg16()
            ci[r8, pl.ds(j * L, L)] = jnp.full((L,), IDX_FILL, jnp.int32)

    iota = lax.iota(jnp.int32, L)

    def make_append(r8, thr):
        def append_one(vj, bidx, cn):
            mask = vj >= thr
            cb = jnp.minimum(cn, CAPL - L)
            plsc.store_compressed(cv.at[r8, pl.ds(cb, L)], vj, mask=mask)
            plsc.store_compressed(ci.at[r8, pl.ds(cb, L)], iota + bidx,
                                  mask=mask)
            return cn + plsc.all_reduce_population_count(mask)[0]
        return append_one

    def procB(buf, w):
        wb = qb + w * WC
        for r8 in range(8):
            thr = jnp.full((L,), thr_smem[r8], jnp.float32)
            append_one = make_append(r8, thr)

            def grp(gi, cn):
                base = gi * (NACC * L)
                vs = [buf[r8, pl.ds(base + j * L, L)] for j in range(NACC)]
                gm = vs[0]
                for j in range(1, NACC):
                    gm = jnp.maximum(gm, vs[j])
                hit = jnp.any(gm >= thr)

                def do(cn2):
                    for j in range(NACC):
                        cn2 = append_one(vs[j], wb + base + j * L, cn2)
                    return cn2

                return lax.cond(hit, do, lambda cn2: cn2, cn)

            cn = lax.fori_loop(0, WC // (NACC * L), grp, cnt_smem[r8])
            cnt_smem[r8] = cn

    run_windows(procB)

    for r8 in range(8):
        obase = pl.multiple_of((blk * 8 + r8) * CAPT + q * CAPL, 8)
        pltpu.sync_copy(cv.at[r8], candv_hbm.at[pl.ds(obase, CAPL)])
        pltpu.sync_copy(ci.at[r8], candi_hbm.at[pl.ds(obase, CAPL)])


@jax.jit
def _sc_topk_candidates(logits):
    mesh = plsc.VectorSubcoreMesh(core_axis_name="c", subcore_axis_name="s",
                                  num_cores=NC, num_subcores=NS)
    f = pl.kernel(
        _sc_body,
        out_type=(jax.ShapeDtypeStruct((B * CAPT,), jnp.float32),
                  jax.ShapeDtypeStruct((B * CAPT,), jnp.int32)),
        mesh=mesh,
        scratch_types=(
            pltpu.VMEM((8, WC), jnp.float32),
            pltpu.VMEM((8, WC), jnp.float32),
            pltpu.VMEM((8, SLOTS), jnp.float32),
            pltpu.VMEM((8, CAPL), jnp.float32),
            pltpu.VMEM((8, CAPL), jnp.int32),
            pltpu.SMEM((8,), jnp.float32),
            pltpu.SMEM((8,), jnp.int32),
            pltpu.SemaphoreType.DMA,
            pltpu.SemaphoreType.DMA,
        ),
        compiler_params=pltpu.CompilerParams(needs_layout_passes=False),
    )
    candv, candi = f(logits)
    return candv.reshape(B, CAPT), candi.reshape(B, CAPT)


def _tc_body(candv_ref, candi_ref, tail_ref, g_ref, probs_ref, samples_ref):
    cv = jnp.concatenate([candv_ref[...], tail_ref[...]], axis=1)
    ci = jnp.concatenate(
        [candi_ref[...],
         QS * SHARD + lax.broadcasted_iota(jnp.int32, (B, TAIL), 1)],
        axis=1)
    colk = lax.broadcasted_iota(jnp.int32, (B, K), 1)

    def body(t, carry):
        cv, tv, ti = carry
        m = jnp.max(cv, axis=1, keepdims=True)
        eq = cv == m
        isel = jnp.min(jnp.where(eq, ci, IDX_FILL), axis=1, keepdims=True)
        cv = jnp.where(eq & (ci == isel), NEGINF, cv)
        tv = jnp.where(colk == t, m, tv)
        ti = jnp.where(colk == t, isel, ti)
        return cv, tv, ti

    _, tv, ti = lax.fori_loop(
        0, K, body,
        (cv, jnp.zeros((B, K), jnp.float32), jnp.zeros((B, K), jnp.int32)))

    mx = jnp.max(tv, axis=1, keepdims=True)
    e = jnp.exp(tv - mx)
    probs = e / jnp.sum(e, axis=1, keepdims=True)
    probs_ref[...] = probs

    score = jnp.log(probs + 1e-20) + g_ref[...]
    smax = jnp.max(score, axis=1, keepdims=True)
    sel = jnp.min(jnp.where(score == smax, colk, IDX_FILL),
                  axis=1, keepdims=True)
    samples_ref[...] = jnp.sum(jnp.where(colk == sel, ti, 0),
                               axis=1, keepdims=True)


def kernel(logits, k):
    candv, candi = _sc_topk_candidates(logits)
    tail = logits[:, QS * SHARD:]
    skey = jax.random.fold_in(jax.random.key(0), 1)
    u = jax.random.uniform(skey, (B, K), dtype=jnp.float32)
    g = -jnp.log(-jnp.log(u + 1e-20) + 1e-20)
    probs, samples = pl.pallas_call(
        _tc_body,
        out_shape=(jax.ShapeDtypeStruct((B, K), jnp.float32),
                   jax.ShapeDtypeStruct((B, 1), jnp.int32)),
    )(candv, candi, tail, g)
    return probs, samples[:, 0]
